# baseline (speedup 1.0000x reference)
import functools
import math

import jax
import jax.numpy as jnp
from jax import lax
from jax.experimental import pallas as pl
from jax.experimental.pallas import tpu as pltpu
from jax.experimental.pallas import tpu_sc as plsc

_B, _C, _H, _W, _K = 16, 10, 256, 256, 256
_HW = _H * _W
_N = _B * _K
_NW = 32
_RPW = _N // _NW
_G = _RPW // 16
_NCH = 8

_ATAN_C = (
    0.9999999981419241, -0.33333292787739005, 0.19998532264290503,
    -0.1426488859280557, 0.10958341276429112, -0.08427560867722841,
    0.058456508004103826, -0.03174908469621797, 0.01125677381196052,
    -0.001877352384963121,
)
_HALF_PI = math.pi / 2.0


def _atan(x):
    ax = jnp.abs(x)
    inv = ax > jnp.float32(1.0)
    t = jnp.where(inv, jnp.float32(1.0) / ax, ax)
    s = t * t
    acc = jnp.full_like(s, jnp.float32(_ATAN_C[-1]))
    for c in reversed(_ATAN_C[:-1]):
        acc = acc * s + jnp.float32(c)
    r = t * acc
    r = jnp.where(inv, jnp.float32(_HALF_PI) - r, r)
    return jnp.where(x < jnp.float32(0.0), -r, r)


def _sc_body(out_hbm, ind_hbm, mask_hbm, tgt_hbm, part_hbm,
             ind_vm, mask_vm, tgt_vm, idx_vm, tidx_vm, pred_vm,
             res_vm, sem):
    wid = lax.axis_index("s") * 2 + lax.axis_index("c")
    r0 = wid * _RPW
    plane0 = (r0 // _K) * (_C * _HW)

    pltpu.sync_copy(ind_hbm.at[pl.ds(r0, _RPW)], ind_vm)
    pltpu.sync_copy(mask_hbm.at[pl.ds(r0, _RPW)], mask_vm)

    iota16 = lax.iota(jnp.int32, 16)
    for g in range(_G):
        ind16 = ind_vm[pl.ds(g * 16, 16)]
        wbase16 = (ind16 >> 1) + jnp.int32(plane0 >> 1)
        trow16 = (iota16 + jnp.int32(g * 16) + r0) * jnp.int32(_C)
        for c in range(_NCH):
            idx_vm[c, pl.ds(g * 16, 16)] = wbase16 + jnp.int32(c * _HW // 2)
            tidx_vm[c, pl.ds(g * 16, 16)] = trow16 + jnp.int32(c)

    copies = [
        pltpu.async_copy(out_hbm.at[idx_vm.at[c]], pred_vm.at[c], sem)
        for c in range(_NCH)
    ] + [
        pltpu.async_copy(tgt_hbm.at[tidx_vm.at[c]], tgt_vm.at[c], sem)
        for c in range(_NCH)
    ]
    for cp in copies:
        cp.wait()

    vacc = jnp.zeros((16,), jnp.float32)
    macc = jnp.zeros((16,), jnp.float32)
    for g in range(_G):
        odd16 = (ind_vm[pl.ds(g * 16, 16)] & jnp.int32(1)) == jnp.int32(1)
        a = []
        for c in range(_NCH):
            wd = pred_vm[c, pl.ds(g * 16, 16)]
            lo = lax.bitcast_convert_type(wd << 16, jnp.float32)
            hi = lax.bitcast_convert_type(wd & jnp.int32(-65536), jnp.float32)
            a.append(jnp.where(odd16, hi, lo))
        b = [tgt_vm[c, pl.ds(g * 16, 16)] for c in range(_NCH)]
        m16 = mask_vm[pl.ds(g * 16, 16)]

        d26, d37 = a[2] - a[6], a[3] - a[7]
        w = d26 * d26 + d37 * d37
        d04, d15 = a[0] - a[4], a[1] - a[5]
        h = d04 * d04 + d15 * d15
        e26, e37a = b[2] - b[6], b[3] - a[7]
        wt = e26 * e26 + e37a * e37a
        e04, e15 = b[0] - b[4], b[1] - b[5]
        ht = e04 * e04 + e15 * e15
        th = d37 / d26
        tth = (b[3] - b[7]) / e26
        da = _atan(wt / ht) - _atan(w / h)
        db = _atan(th) - _atan(tth)
        v = jnp.float32(4.0 / math.pi**2) * (da * da + db * db)
        vacc = vacc + v * m16
        macc = macc + m16

    res_vm[pl.ds(0, 16)] = vacc
    res_vm[pl.ds(16, 16)] = macc
    pltpu.sync_copy(res_vm, part_hbm.at[pl.ds(wid * 32, 32)])


@jax.jit
def _sc_loss(out_flat, ind_flat, mask_flat, tgt_rows):
    mesh = plsc.VectorSubcoreMesh(core_axis_name="c", subcore_axis_name="s")
    parts = pl.kernel(
        _sc_body,
        out_type=jax.ShapeDtypeStruct((_NW * 32,), jnp.float32),
        mesh=mesh,
        scratch_types=[
            pltpu.VMEM((_RPW,), jnp.int32),
            pltpu.VMEM((_RPW,), jnp.float32),
            pltpu.VMEM((_NCH, _RPW), jnp.float32),
            pltpu.VMEM((_NCH, _RPW), jnp.int32),
            pltpu.VMEM((_NCH, _RPW), jnp.int32),
            pltpu.VMEM((_NCH, _RPW), jnp.int32),
            pltpu.VMEM((32,), jnp.float32),
            pltpu.SemaphoreType.DMA,
        ],
    )(out_flat, ind_flat, mask_flat, tgt_rows)
    return parts


def kernel(output, mask, ind, target):
    out_flat = jax.lax.bitcast_convert_type(
        output.astype(jnp.bfloat16).reshape(-1, 2), jnp.int32)
    ind_flat = ind.reshape(-1).astype(jnp.int32)
    mask_flat = mask.reshape(-1)
    tgt_flat = target.reshape(-1)
    parts = _sc_loss(out_flat, ind_flat, mask_flat, tgt_flat).reshape(_NW, 2, 16)
    loss = jnp.sum(parts[:, 0, :])
    msum = jnp.sum(parts[:, 1, :])
    return jnp.where(msum == 0.0, jnp.float32(0.0), loss)


# speedup vs baseline: 30.9498x; 30.9498x over previous
import functools
import math

import jax
import jax.numpy as jnp
from jax import lax
from jax.experimental import pallas as pl
from jax.experimental.pallas import tpu as pltpu
from jax.experimental.pallas import tpu_sc as plsc

_B, _C, _H, _W, _K = 16, 10, 256, 256, 256
_HW = _H * _W
_N = _B * _K
_NW = 32
_RPW = _N // _NW
_G = _RPW // 16
_NCH = 8

_ATAN_C = (
    0.9999999981419241, -0.33333292787739005, 0.19998532264290503,
    -0.1426488859280557, 0.10958341276429112, -0.08427560867722841,
    0.058456508004103826, -0.03174908469621797, 0.01125677381196052,
    -0.001877352384963121,
)
_HALF_PI = math.pi / 2.0


def _atan(x):
    ax = jnp.abs(x)
    inv = ax > jnp.float32(1.0)
    t = jnp.where(inv, jnp.float32(1.0) / ax, ax)
    s = t * t
    acc = jnp.full_like(s, jnp.float32(_ATAN_C[-1]))
    for c in reversed(_ATAN_C[:-1]):
        acc = acc * s + jnp.float32(c)
    r = t * acc
    r = jnp.where(inv, jnp.float32(_HALF_PI) - r, r)
    return jnp.where(x < jnp.float32(0.0), -r, r)


_WPP = _HW // 2


def _pack_body(x_ref, o_ref):
    u = lax.bitcast_convert_type(x_ref[0, 0], jnp.uint32)
    r = (u + jnp.uint32(0x7FFF) + ((u >> 16) & jnp.uint32(1))) >> 16
    word = r[:, :128] | (r[:, 128:] << 16)
    o_ref[...] = lax.bitcast_convert_type(word, jnp.int32)


@jax.jit
def _pack(output):
    return pl.pallas_call(
        _pack_body,
        grid=(_B * _C,),
        in_specs=[pl.BlockSpec((1, 1, _H, _W),
                               lambda p: (p // _C, p % _C, 0, 0))],
        out_specs=pl.BlockSpec((_H, _W // 2), lambda p: (p, 0)),
        out_shape=jax.ShapeDtypeStruct((_B * _C * _H, _W // 2), jnp.int32),
    )(output)


def _sc_body(out_hbm, ind_hbm, mask_hbm, tgt_hbm, part_hbm,
             ind_vm, mask_vm, tgt_vm, idx_vm, tidx_vm, pred_vm,
             res_vm, sem):
    wid = lax.axis_index("s") * 2 + lax.axis_index("c")
    r0 = wid * _RPW
    plane0 = (r0 // _K) * (_C * _HW)

    pltpu.sync_copy(ind_hbm.at[pl.ds(r0, _RPW)], ind_vm)
    pltpu.sync_copy(mask_hbm.at[pl.ds(r0, _RPW)], mask_vm)

    iota16 = lax.iota(jnp.int32, 16)
    for g in range(_G):
        ind16 = ind_vm[pl.ds(g * 16, 16)]
        wbase16 = (((ind16 >> 8) << 7) | (ind16 & jnp.int32(127))
                   ) + jnp.int32(plane0 >> 1)
        trow16 = (iota16 + jnp.int32(g * 16) + r0) * jnp.int32(_C)
        for c in range(_NCH):
            idx_vm[c, pl.ds(g * 16, 16)] = wbase16 + jnp.int32(c * _WPP)
            tidx_vm[c, pl.ds(g * 16, 16)] = trow16 + jnp.int32(c)

    copies = [
        pltpu.async_copy(out_hbm.at[idx_vm.at[c]], pred_vm.at[c], sem)
        for c in range(_NCH)
    ] + [
        pltpu.async_copy(tgt_hbm.at[tidx_vm.at[c]], tgt_vm.at[c], sem)
        for c in range(_NCH)
    ]
    for cp in copies:
        cp.wait()

    vacc = jnp.zeros((16,), jnp.float32)
    macc = jnp.zeros((16,), jnp.float32)
    for g in range(_G):
        odd16 = (ind_vm[pl.ds(g * 16, 16)] & jnp.int32(128)) == jnp.int32(128)
        a = []
        for c in range(_NCH):
            wd = pred_vm[c, pl.ds(g * 16, 16)]
            lo = lax.bitcast_convert_type(wd << 16, jnp.float32)
            hi = lax.bitcast_convert_type(wd & jnp.int32(-65536), jnp.float32)
            a.append(jnp.where(odd16, hi, lo))
        b = [tgt_vm[c, pl.ds(g * 16, 16)] for c in range(_NCH)]
        m16 = mask_vm[pl.ds(g * 16, 16)]

        d26, d37 = a[2] - a[6], a[3] - a[7]
        w = d26 * d26 + d37 * d37
        d04, d15 = a[0] - a[4], a[1] - a[5]
        h = d04 * d04 + d15 * d15
        e26, e37a = b[2] - b[6], b[3] - a[7]
        wt = e26 * e26 + e37a * e37a
        e04, e15 = b[0] - b[4], b[1] - b[5]
        ht = e04 * e04 + e15 * e15
        th = d37 / d26
        tth = (b[3] - b[7]) / e26
        da = _atan(wt / ht) - _atan(w / h)
        db = _atan(th) - _atan(tth)
        v = jnp.float32(4.0 / math.pi**2) * (da * da + db * db)
        vacc = vacc + v * m16
        macc = macc + m16

    res_vm[pl.ds(0, 16)] = vacc
    res_vm[pl.ds(16, 16)] = macc
    pltpu.sync_copy(res_vm, part_hbm.at[pl.ds(wid * 32, 32)])


@jax.jit
def _sc_loss(out_flat, ind_flat, mask_flat, tgt_rows):
    mesh = plsc.VectorSubcoreMesh(core_axis_name="c", subcore_axis_name="s")
    parts = pl.kernel(
        _sc_body,
        out_type=jax.ShapeDtypeStruct((_NW * 32,), jnp.float32),
        mesh=mesh,
        scratch_types=[
            pltpu.VMEM((_RPW,), jnp.int32),
            pltpu.VMEM((_RPW,), jnp.float32),
            pltpu.VMEM((_NCH, _RPW), jnp.float32),
            pltpu.VMEM((_NCH, _RPW), jnp.int32),
            pltpu.VMEM((_NCH, _RPW), jnp.int32),
            pltpu.VMEM((_NCH, _RPW), jnp.int32),
            pltpu.VMEM((32,), jnp.float32),
            pltpu.SemaphoreType.DMA,
        ],
    )(out_flat, ind_flat, mask_flat, tgt_rows)
    return parts


def kernel(output, mask, ind, target):
    out_flat = _pack(output).reshape(-1)
    ind_flat = ind.reshape(-1).astype(jnp.int32)
    mask_flat = mask.reshape(-1)
    tgt_flat = target.reshape(-1)
    parts = _sc_loss(out_flat, ind_flat, mask_flat, tgt_flat).reshape(_NW, 2, 16)
    loss = jnp.sum(parts[:, 0, :])
    msum = jnp.sum(parts[:, 1, :])
    return jnp.where(msum == 0.0, jnp.float32(0.0), loss)


# speedup vs baseline: 68.0108x; 2.1975x over previous
import functools
import math

import jax
import jax.numpy as jnp
from jax import lax
from jax.experimental import pallas as pl
from jax.experimental.pallas import tpu as pltpu
from jax.experimental.pallas import tpu_sc as plsc

_B, _C, _H, _W, _K = 16, 10, 256, 256, 256
_HW = _H * _W
_N = _B * _K
_NW = 32
_RPW = _N // _NW
_G = _RPW // 16
_NCH = 8

_ATAN_C = (
    0.9999999981419241, -0.33333292787739005, 0.19998532264290503,
    -0.1426488859280557, 0.10958341276429112, -0.08427560867722841,
    0.058456508004103826, -0.03174908469621797, 0.01125677381196052,
    -0.001877352384963121,
)
_HALF_PI = math.pi / 2.0


def _atan(x):
    ax = jnp.abs(x)
    inv = ax > jnp.float32(1.0)
    t = jnp.where(inv, jnp.float32(1.0) / ax, ax)
    s = t * t
    acc = jnp.full_like(s, jnp.float32(_ATAN_C[-1]))
    for c in reversed(_ATAN_C[:-1]):
        acc = acc * s + jnp.float32(c)
    r = t * acc
    r = jnp.where(inv, jnp.float32(_HALF_PI) - r, r)
    return jnp.where(x < jnp.float32(0.0), -r, r)


_WPP = _HW // 2


def _pack_body(x_ref, o_ref):
    u = lax.bitcast_convert_type(x_ref[0], jnp.uint32)
    r = (u + jnp.uint32(0x7FFF) + ((u >> 16) & jnp.uint32(1))) >> 16
    word = r[:, :, :128] | (r[:, :, 128:] << 16)
    o_ref[...] = lax.bitcast_convert_type(
        word, jnp.int32).reshape(_C * _H, _W // 2)


@jax.jit
def _pack(output):
    return pl.pallas_call(
        _pack_body,
        grid=(_B,),
        in_specs=[pl.BlockSpec((1, _C, _H, _W), lambda p: (p, 0, 0, 0))],
        out_specs=pl.BlockSpec((_C * _H, _W // 2), lambda p: (p, 0)),
        out_shape=jax.ShapeDtypeStruct((_B * _C * _H, _W // 2), jnp.int32),
    )(output)


def _sc_body(out_hbm, ind_hbm, mask_hbm, tgt_hbm, part_hbm,
             ind_vm, mask_vm, tgt_vm, idx_vm, tidx_vm, pred_vm,
             res_vm, sem):
    wid = lax.axis_index("s") * 2 + lax.axis_index("c")
    r0 = wid * _RPW
    plane0 = (r0 // _K) * (_C * _HW)

    pltpu.sync_copy(ind_hbm.at[pl.ds(r0, _RPW)], ind_vm)
    pltpu.sync_copy(mask_hbm.at[pl.ds(r0, _RPW)], mask_vm)

    iota16 = lax.iota(jnp.int32, 16)
    for g in range(_G):
        ind16 = ind_vm[pl.ds(g * 16, 16)]
        wbase16 = (((ind16 >> 8) << 7) | (ind16 & jnp.int32(127))
                   ) + jnp.int32(plane0 >> 1)
        trow16 = (iota16 + jnp.int32(g * 16) + r0) * jnp.int32(_C)
        for c in range(_NCH):
            idx_vm[c, pl.ds(g * 16, 16)] = wbase16 + jnp.int32(c * _WPP)
            tidx_vm[c, pl.ds(g * 16, 16)] = trow16 + jnp.int32(c)

    copies = [
        pltpu.async_copy(out_hbm.at[idx_vm.at[c]], pred_vm.at[c], sem)
        for c in range(_NCH)
    ] + [
        pltpu.async_copy(tgt_hbm.at[tidx_vm.at[c]], tgt_vm.at[c], sem)
        for c in range(_NCH)
    ]
    for cp in copies:
        cp.wait()

    vacc = jnp.zeros((16,), jnp.float32)
    macc = jnp.zeros((16,), jnp.float32)
    for g in range(_G):
        odd16 = (ind_vm[pl.ds(g * 16, 16)] & jnp.int32(128)) == jnp.int32(128)
        a = []
        for c in range(_NCH):
            wd = pred_vm[c, pl.ds(g * 16, 16)]
            lo = lax.bitcast_convert_type(wd << 16, jnp.float32)
            hi = lax.bitcast_convert_type(wd & jnp.int32(-65536), jnp.float32)
            a.append(jnp.where(odd16, hi, lo))
        b = [tgt_vm[c, pl.ds(g * 16, 16)] for c in range(_NCH)]
        m16 = mask_vm[pl.ds(g * 16, 16)]

        d26, d37 = a[2] - a[6], a[3] - a[7]
        w = d26 * d26 + d37 * d37
        d04, d15 = a[0] - a[4], a[1] - a[5]
        h = d04 * d04 + d15 * d15
        e26, e37a = b[2] - b[6], b[3] - a[7]
        wt = e26 * e26 + e37a * e37a
        e04, e15 = b[0] - b[4], b[1] - b[5]
        ht = e04 * e04 + e15 * e15
        th = d37 / d26
        tth = (b[3] - b[7]) / e26
        da = _atan(wt / ht) - _atan(w / h)
        db = _atan(th) - _atan(tth)
        v = jnp.float32(4.0 / math.pi**2) * (da * da + db * db)
        vacc = vacc + v * m16
        macc = macc + m16

    res_vm[pl.ds(0, 16)] = vacc
    res_vm[pl.ds(16, 16)] = macc
    pltpu.sync_copy(res_vm, part_hbm.at[pl.ds(wid * 32, 32)])


@jax.jit
def _sc_loss(out_flat, ind_flat, mask_flat, tgt_rows):
    mesh = plsc.VectorSubcoreMesh(core_axis_name="c", subcore_axis_name="s")
    parts = pl.kernel(
        _sc_body,
        out_type=jax.ShapeDtypeStruct((_NW * 32,), jnp.float32),
        mesh=mesh,
        scratch_types=[
            pltpu.VMEM((_RPW,), jnp.int32),
            pltpu.VMEM((_RPW,), jnp.float32),
            pltpu.VMEM((_NCH, _RPW), jnp.float32),
            pltpu.VMEM((_NCH, _RPW), jnp.int32),
            pltpu.VMEM((_NCH, _RPW), jnp.int32),
            pltpu.VMEM((_NCH, _RPW), jnp.int32),
            pltpu.VMEM((32,), jnp.float32),
            pltpu.SemaphoreType.DMA,
        ],
    )(out_flat, ind_flat, mask_flat, tgt_rows)
    return parts


def kernel(output, mask, ind, target):
    out_flat = _pack(output).reshape(-1)
    ind_flat = ind.reshape(-1).astype(jnp.int32)
    mask_flat = mask.reshape(-1)
    tgt_flat = target.reshape(-1)
    parts = _sc_loss(out_flat, ind_flat, mask_flat, tgt_flat).reshape(_NW, 2, 16)
    loss = jnp.sum(parts[:, 0, :])
    msum = jnp.sum(parts[:, 1, :])
    return jnp.where(msum == 0.0, jnp.float32(0.0), loss)


# speedup vs baseline: 69.8931x; 1.0277x over previous
import functools
import math

import jax
import jax.numpy as jnp
from jax import lax
from jax.experimental import pallas as pl
from jax.experimental.pallas import tpu as pltpu
from jax.experimental.pallas import tpu_sc as plsc

_B, _C, _H, _W, _K = 16, 10, 256, 256, 256
_HW = _H * _W
_N = _B * _K
_NW = 32
_RPW = _N // _NW
_G = _RPW // 16
_NCH = 8

_ATAN_C = (
    0.9999999981419241, -0.33333292787739005, 0.19998532264290503,
    -0.1426488859280557, 0.10958341276429112, -0.08427560867722841,
    0.058456508004103826, -0.03174908469621797, 0.01125677381196052,
    -0.001877352384963121,
)
_HALF_PI = math.pi / 2.0


def _atan(x):
    ax = jnp.abs(x)
    inv = ax > jnp.float32(1.0)
    t = jnp.where(inv, jnp.float32(1.0) / ax, ax)
    s = t * t
    acc = jnp.full_like(s, jnp.float32(_ATAN_C[-1]))
    for c in reversed(_ATAN_C[:-1]):
        acc = acc * s + jnp.float32(c)
    r = t * acc
    r = jnp.where(inv, jnp.float32(_HALF_PI) - r, r)
    return jnp.where(x < jnp.float32(0.0), -r, r)


_WPP = _HW // 2


def _pack_body(x_ref, o_ref):
    u = lax.bitcast_convert_type(x_ref[0], jnp.uint32)
    r = (u + jnp.uint32(0x8000)) >> 16
    word = r[:, :, :128] | (r[:, :, 128:] << 16)
    o_ref[...] = lax.bitcast_convert_type(
        word, jnp.int32).reshape(_C * _H, _W // 2)


@jax.jit
def _pack(output):
    return pl.pallas_call(
        _pack_body,
        grid=(_B,),
        in_specs=[pl.BlockSpec((1, _C, _H, _W), lambda p: (p, 0, 0, 0))],
        out_specs=pl.BlockSpec((_C * _H, _W // 2), lambda p: (p, 0)),
        out_shape=jax.ShapeDtypeStruct((_B * _C * _H, _W // 2), jnp.int32),
    )(output)


def _sc_body(out_hbm, ind_hbm, mask_hbm, tgt_hbm, part_hbm,
             ind_vm, mask_vm, tgt_vm, idx_vm, tidx_vm, pred_vm,
             res_vm, sem):
    wid = lax.axis_index("s") * 2 + lax.axis_index("c")
    r0 = wid * _RPW
    plane0 = (r0 // _K) * (_C * _HW)

    pltpu.sync_copy(ind_hbm.at[pl.ds(r0, _RPW)], ind_vm)
    pltpu.sync_copy(mask_hbm.at[pl.ds(r0, _RPW)], mask_vm)

    iota16 = lax.iota(jnp.int32, 16)
    for g in range(_G):
        ind16 = ind_vm[pl.ds(g * 16, 16)]
        wbase16 = (((ind16 >> 8) << 7) | (ind16 & jnp.int32(127))
                   ) + jnp.int32(plane0 >> 1)
        trow16 = (iota16 + jnp.int32(g * 16) + r0) * jnp.int32(_C)
        for c in range(_NCH):
            idx_vm[c, pl.ds(g * 16, 16)] = wbase16 + jnp.int32(c * _WPP)
            tidx_vm[c, pl.ds(g * 16, 16)] = trow16 + jnp.int32(c)

    copies = [
        pltpu.async_copy(out_hbm.at[idx_vm.at[c]], pred_vm.at[c], sem)
        for c in range(_NCH)
    ] + [
        pltpu.async_copy(tgt_hbm.at[tidx_vm.at[c]], tgt_vm.at[c], sem)
        for c in range(_NCH)
    ]
    for cp in copies:
        cp.wait()

    vacc = jnp.zeros((16,), jnp.float32)
    macc = jnp.zeros((16,), jnp.float32)
    for g in range(_G):
        odd16 = (ind_vm[pl.ds(g * 16, 16)] & jnp.int32(128)) == jnp.int32(128)
        a = []
        for c in range(_NCH):
            wd = pred_vm[c, pl.ds(g * 16, 16)]
            lo = lax.bitcast_convert_type(wd << 16, jnp.float32)
            hi = lax.bitcast_convert_type(wd & jnp.int32(-65536), jnp.float32)
            a.append(jnp.where(odd16, hi, lo))
        b = [tgt_vm[c, pl.ds(g * 16, 16)] for c in range(_NCH)]
        m16 = mask_vm[pl.ds(g * 16, 16)]

        d26, d37 = a[2] - a[6], a[3] - a[7]
        w = d26 * d26 + d37 * d37
        d04, d15 = a[0] - a[4], a[1] - a[5]
        h = d04 * d04 + d15 * d15
        e26, e37a = b[2] - b[6], b[3] - a[7]
        wt = e26 * e26 + e37a * e37a
        e04, e15 = b[0] - b[4], b[1] - b[5]
        ht = e04 * e04 + e15 * e15
        th = d37 / d26
        tth = (b[3] - b[7]) / e26
        da = _atan(wt / ht) - _atan(w / h)
        db = _atan(th) - _atan(tth)
        v = jnp.float32(4.0 / math.pi**2) * (da * da + db * db)
        vacc = vacc + v * m16
        macc = macc + m16

    res_vm[pl.ds(0, 16)] = vacc
    res_vm[pl.ds(16, 16)] = macc
    pltpu.sync_copy(res_vm, part_hbm.at[pl.ds(wid * 32, 32)])


@jax.jit
def _sc_loss(out_flat, ind_flat, mask_flat, tgt_rows):
    mesh = plsc.VectorSubcoreMesh(core_axis_name="c", subcore_axis_name="s")
    parts = pl.kernel(
        _sc_body,
        out_type=jax.ShapeDtypeStruct((_NW * 32,), jnp.float32),
        mesh=mesh,
        scratch_types=[
            pltpu.VMEM((_RPW,), jnp.int32),
            pltpu.VMEM((_RPW,), jnp.float32),
            pltpu.VMEM((_NCH, _RPW), jnp.float32),
            pltpu.VMEM((_NCH, _RPW), jnp.int32),
            pltpu.VMEM((_NCH, _RPW), jnp.int32),
            pltpu.VMEM((_NCH, _RPW), jnp.int32),
            pltpu.VMEM((32,), jnp.float32),
            pltpu.SemaphoreType.DMA,
        ],
    )(out_flat, ind_flat, mask_flat, tgt_rows)
    return parts


def kernel(output, mask, ind, target):
    out_flat = _pack(output).reshape(-1)
    ind_flat = ind.reshape(-1).astype(jnp.int32)
    mask_flat = mask.reshape(-1)
    tgt_flat = target.reshape(-1)
    parts = _sc_loss(out_flat, ind_flat, mask_flat, tgt_flat).reshape(_NW, 2, 16)
    loss = jnp.sum(parts[:, 0, :])
    msum = jnp.sum(parts[:, 1, :])
    return jnp.where(msum == 0.0, jnp.float32(0.0), loss)


# speedup vs baseline: 77.9997x; 1.1160x over previous
import functools
import math

import jax
import jax.numpy as jnp
from jax import lax
from jax.experimental import pallas as pl
from jax.experimental.pallas import tpu as pltpu
from jax.experimental.pallas import tpu_sc as plsc

_B, _C, _H, _W, _K = 16, 10, 256, 256, 256
_HW = _H * _W
_N = _B * _K
_NW = 32
_RPW = _N // _NW
_G = _RPW // 16
_NCH = 8

_ATAN_C = (
    0.9999999981419241, -0.33333292787739005, 0.19998532264290503,
    -0.1426488859280557, 0.10958341276429112, -0.08427560867722841,
    0.058456508004103826, -0.03174908469621797, 0.01125677381196052,
    -0.001877352384963121,
)
_HALF_PI = math.pi / 2.0


def _atan(x):
    ax = jnp.abs(x)
    inv = ax > jnp.float32(1.0)
    t = jnp.where(inv, jnp.float32(1.0) / ax, ax)
    s = t * t
    acc = jnp.full_like(s, jnp.float32(_ATAN_C[-1]))
    for c in reversed(_ATAN_C[:-1]):
        acc = acc * s + jnp.float32(c)
    r = t * acc
    r = jnp.where(inv, jnp.float32(_HALF_PI) - r, r)
    return jnp.where(x < jnp.float32(0.0), -r, r)


_WPP = _HW // 2


def _pack_body(x_ref, t_ref, o_ref, t8_ref):
    u = lax.bitcast_convert_type(x_ref[0], jnp.uint32)
    r = (u + jnp.uint32(0x8000)) >> 16
    word = r[:, :, :128] | (r[:, :, 128:] << 16)
    o_ref[...] = lax.bitcast_convert_type(
        word, jnp.int32).reshape(_NCH * _H, _W // 2)
    t8_ref[...] = jnp.transpose(t_ref[0, :, :_NCH], (1, 0))


@jax.jit
def _pack(output, target):
    return pl.pallas_call(
        _pack_body,
        grid=(_B,),
        in_specs=[
            pl.BlockSpec((1, _NCH, _H, _W), lambda p: (p, 0, 0, 0)),
            pl.BlockSpec((1, _K, _C), lambda p: (p, 0, 0)),
        ],
        out_specs=[
            pl.BlockSpec((_NCH * _H, _W // 2), lambda p: (p, 0)),
            pl.BlockSpec((_NCH, _K), lambda p: (0, p)),
        ],
        out_shape=[
            jax.ShapeDtypeStruct((_B * _NCH * _H, _W // 2), jnp.int32),
            jax.ShapeDtypeStruct((_NCH, _N), jnp.float32),
        ],
    )(output, target)


def _sc_body(out_hbm, ind_hbm, mask_hbm, tgt_hbm, part_hbm,
             ind8_vm, mask8_vm, tgt_vm, idx_vm, pred_vm,
             res_vm, sem):
    wid = lax.axis_index("s") * 2 + lax.axis_index("c")
    r0 = wid * _RPW
    b = r0 // _K
    lb = b // 8
    br = b % 8
    k0 = (wid % 2) * _RPW
    plane0w = b * (_NCH * _WPP)

    pltpu.sync_copy(ind_hbm.at[lb, :, pl.ds(k0, _RPW)], ind8_vm)
    pltpu.sync_copy(mask_hbm.at[lb, :, pl.ds(k0, _RPW)], mask8_vm)
    pltpu.sync_copy(tgt_hbm.at[:, pl.ds(r0, _RPW)], tgt_vm)

    for g in range(_G):
        ind16 = ind8_vm[br, pl.ds(g * 16, 16)]
        wbase16 = (((ind16 >> 8) << 7) | (ind16 & jnp.int32(127))
                   ) + jnp.int32(plane0w)
        for c in range(_NCH):
            idx_vm[c, pl.ds(g * 16, 16)] = wbase16 + jnp.int32(c * _WPP)

    copies = [
        pltpu.async_copy(out_hbm.at[idx_vm.at[c]], pred_vm.at[c], sem)
        for c in range(_NCH)
    ]
    for cp in copies:
        cp.wait()

    vacc = jnp.zeros((16,), jnp.float32)
    macc = jnp.zeros((16,), jnp.float32)
    for g in range(_G):
        odd16 = (ind8_vm[br, pl.ds(g * 16, 16)] & jnp.int32(128)
                 ) == jnp.int32(128)
        a = []
        for c in range(_NCH):
            wd = pred_vm[c, pl.ds(g * 16, 16)]
            lo = lax.bitcast_convert_type(wd << 16, jnp.float32)
            hi = lax.bitcast_convert_type(wd & jnp.int32(-65536), jnp.float32)
            a.append(jnp.where(odd16, hi, lo))
        b = [tgt_vm[c, pl.ds(g * 16, 16)] for c in range(_NCH)]
        m16 = mask8_vm[br, pl.ds(g * 16, 16)]

        d26, d37 = a[2] - a[6], a[3] - a[7]
        w = d26 * d26 + d37 * d37
        d04, d15 = a[0] - a[4], a[1] - a[5]
        h = d04 * d04 + d15 * d15
        e26, e37a = b[2] - b[6], b[3] - a[7]
        wt = e26 * e26 + e37a * e37a
        e04, e15 = b[0] - b[4], b[1] - b[5]
        ht = e04 * e04 + e15 * e15
        th = d37 / d26
        tth = (b[3] - b[7]) / e26
        da = _atan(wt / ht) - _atan(w / h)
        db = _atan(th) - _atan(tth)
        v = jnp.float32(4.0 / math.pi**2) * (da * da + db * db)
        vacc = vacc + v * m16
        macc = macc + m16

    res_vm[pl.ds(0, 16)] = vacc
    res_vm[pl.ds(16, 16)] = macc
    pltpu.sync_copy(res_vm, part_hbm.at[pl.ds(wid * 32, 32)])


@jax.jit
def _sc_loss(out_flat, ind_flat, mask_flat, tgt_rows):
    mesh = plsc.VectorSubcoreMesh(core_axis_name="c", subcore_axis_name="s")
    parts = pl.kernel(
        _sc_body,
        out_type=jax.ShapeDtypeStruct((_NW * 32,), jnp.float32),
        mesh=mesh,
        scratch_types=[
            pltpu.VMEM((8, _RPW), jnp.int32),
            pltpu.VMEM((8, _RPW), jnp.float32),
            pltpu.VMEM((_NCH, _RPW), jnp.float32),
            pltpu.VMEM((_NCH, _RPW), jnp.int32),
            pltpu.VMEM((_NCH, _RPW), jnp.int32),
            pltpu.VMEM((32,), jnp.float32),
            pltpu.SemaphoreType.DMA,
        ],
    )(out_flat, ind_flat, mask_flat, tgt_rows)
    return parts


def kernel(output, mask, ind, target):
    words, tgt8 = _pack(output, target)
    out_flat = words.reshape(-1)
    ind3 = ind.astype(jnp.int32).reshape(2, 8, _K)
    mask3 = mask.reshape(2, 8, _K)
    parts = _sc_loss(out_flat, ind3, mask3, tgt8).reshape(_NW, 2, 16)
    loss = jnp.sum(parts[:, 0, :])
    msum = jnp.sum(parts[:, 1, :])
    return jnp.where(msum == 0.0, jnp.float32(0.0), loss)
